# baseline (speedup 1.0000x reference)
import jax
import jax.numpy as jnp
from jax.experimental import pallas as pl
from jax.experimental.pallas import tpu as pltpu


def _round_up(x, m):
    return ((x + m - 1) // m) * m


def _conv_stats_kernel(x_ref, b_ref, y_ref, s_ref, ss_ref):
    nb, cin, hw = x_ref.shape
    lanes = b_ref.shape[1]
    h = hw // (lanes // cin - 2)
    w = hw // h
    xb = x_ref[...].astype(jnp.bfloat16)
    rows = jnp.transpose(xb.reshape(nb, cin, h, w), (0, 2, 3, 1)).reshape(
        nb, h, w * cin)
    zl = jnp.zeros((nb, h, cin), jnp.bfloat16)
    rows = jnp.concatenate([zl, rows, zl], axis=2)
    zr = jnp.zeros((nb, 1, lanes), jnp.bfloat16)
    rows = jnp.concatenate([zr, rows, zr], axis=1)
    acc = jnp.dot(rows[:, 0:h, :].reshape(nb * h, lanes), b_ref[0],
                  preferred_element_type=jnp.float32)
    acc = acc + jnp.dot(rows[:, 1:h + 1, :].reshape(nb * h, lanes), b_ref[1],
                        preferred_element_type=jnp.float32)
    acc = acc + jnp.dot(rows[:, 2:h + 2, :].reshape(nb * h, lanes), b_ref[2],
                        preferred_element_type=jnp.float32)
    y_ref[...] = acc.astype(y_ref.dtype)
    s_ref[0] = jnp.sum(acc, axis=0, keepdims=True)
    ss_ref[0] = jnp.sum(acc * acc, axis=0, keepdims=True)


def _bn_relu_kernel(y_ref, scale_ref, shift_ref, o_ref):
    nb, cout, hw = o_ref.shape
    h = y_ref.shape[0] // nb
    w = hw // h
    y = y_ref[...].astype(jnp.float32)
    o = jnp.maximum(y * scale_ref[...] + shift_ref[...], 0.0)
    o = o[:, :w * cout].reshape(nb, h, w, cout)
    o_ref[...] = jnp.transpose(o, (0, 3, 1, 2)).reshape(nb, cout, hw)


def kernel(x_nchw, w_oihw, gamma, beta, eps=1e-5):
    N, Cin, H, W = x_nchw.shape
    Cout, Cin2, KH, KW = w_oihw.shape
    assert Cin2 == Cin and KH == 3 and KW == 3

    Wp = W + 2
    lanes = Wp * Cin
    Lout = W * Cout
    LoutP = _round_up(Lout, 128)
    M = N * H

    x3 = x_nchw.reshape(N, Cin, H * W)

    wt = jnp.transpose(w_oihw, (2, 3, 1, 0)).astype(jnp.float32)
    dx = jnp.arange(KW)[:, None, None]
    xw = jnp.arange(Wp)[None, :, None]
    wv = jnp.arange(W)[None, None, :]
    S = (xw == wv + dx).astype(jnp.float32)
    B = jnp.einsum('apw,daco->dpcwo', S, wt).reshape(KH, lanes, Lout)
    if LoutP != Lout:
        B = jnp.pad(B, ((0, 0), (0, 0), (0, LoutP - Lout)))
    B = B.astype(jnp.bfloat16)

    nb = 8
    while N % nb:
        nb //= 2
    T = N // nb

    cparams = pltpu.CompilerParams(
        dimension_semantics=("parallel",),
        vmem_limit_bytes=96 * 1024 * 1024,
    )

    conv_y, psum, psumsq = pl.pallas_call(
        _conv_stats_kernel,
        grid=(T,),
        out_shape=(
            jax.ShapeDtypeStruct((M, LoutP), jnp.bfloat16),
            jax.ShapeDtypeStruct((T, 1, LoutP), jnp.float32),
            jax.ShapeDtypeStruct((T, 1, LoutP), jnp.float32),
        ),
        in_specs=[
            pl.BlockSpec((nb, Cin, H * W), lambda i: (i, 0, 0)),
            pl.BlockSpec((KH, lanes, LoutP), lambda i: (0, 0, 0)),
        ],
        out_specs=(
            pl.BlockSpec((nb * H, LoutP), lambda i: (i, 0)),
            pl.BlockSpec((1, 1, LoutP), lambda i: (i, 0, 0)),
            pl.BlockSpec((1, 1, LoutP), lambda i: (i, 0, 0)),
        ),
        compiler_params=cparams,
        cost_estimate=pl.CostEstimate(
            flops=2 * M * KH * lanes * LoutP,
            transcendentals=0,
            bytes_accessed=4 * N * Cin * H * W + 2 * (KH * lanes * LoutP
                           + M * LoutP) + 8 * T * LoutP,
        ),
    )(x3, B)

    count = N * H * W
    lane_sum = jnp.sum(psum, axis=(0, 1))[:Lout]
    lane_sumsq = jnp.sum(psumsq, axis=(0, 1))[:Lout]
    ch_sum = lane_sum.reshape(W, Cout).sum(axis=0)
    ch_sumsq = lane_sumsq.reshape(W, Cout).sum(axis=0)
    mean = ch_sum / count
    var = jnp.maximum(ch_sumsq / count - mean * mean, 0.0)
    inv_std = jax.lax.rsqrt(var + eps)
    scale_c = gamma.astype(jnp.float32) * inv_std
    shift_c = beta.astype(jnp.float32) - mean * scale_c
    scale_v = jnp.tile(scale_c, W).reshape(1, Lout)
    shift_v = jnp.tile(shift_c, W).reshape(1, Lout)
    if LoutP != Lout:
        scale_v = jnp.pad(scale_v, ((0, 0), (0, LoutP - Lout)))
        shift_v = jnp.pad(shift_v, ((0, 0), (0, LoutP - Lout)))

    out3 = pl.pallas_call(
        _bn_relu_kernel,
        grid=(T,),
        out_shape=jax.ShapeDtypeStruct((N, Cout, H * W), jnp.float32),
        in_specs=[
            pl.BlockSpec((nb * H, LoutP), lambda i: (i, 0)),
            pl.BlockSpec((1, LoutP), lambda i: (0, 0)),
            pl.BlockSpec((1, LoutP), lambda i: (0, 0)),
        ],
        out_specs=pl.BlockSpec((nb, Cout, H * W), lambda i: (i, 0, 0)),
        compiler_params=cparams,
        cost_estimate=pl.CostEstimate(
            flops=2 * M * LoutP,
            transcendentals=0,
            bytes_accessed=6 * M * LoutP,
        ),
    )(conv_y, scale_v, shift_v)

    return out3.reshape(N, Cout, H, W)


# speedup vs baseline: 2.6389x; 2.6389x over previous
import jax
import jax.numpy as jnp
from jax.experimental import pallas as pl
from jax.experimental.pallas import tpu as pltpu


def _conv_stats_kernel(xm_ref, xu_ref, xd_ref, b_ref, y_ref, s_ref, ss_ref):
    cin, th, w, n = xm_ref.shape
    k = cin * w
    i = pl.program_id(0)
    t = pl.num_programs(0)
    xm = xm_ref[...].astype(jnp.bfloat16)
    up = (jnp.where(i == 0, 0.0, 1.0) * xu_ref[...]).astype(jnp.bfloat16)
    dn = (jnp.where(i == t - 1, 0.0, 1.0) * xd_ref[...]).astype(jnp.bfloat16)
    cols = [up.reshape(k, n)]
    for h in range(th):
        cols.append(xm[:, h].reshape(k, n))
    cols.append(dn.reshape(k, n))
    a = jnp.concatenate(cols, axis=1)
    acc = None
    for dy in range(3):
        lhs = jax.lax.slice(a, (0, dy * n), (k, (dy + th) * n))
        p = jax.lax.dot_general(lhs, b_ref[dy], (((0,), (0,)), ((), ())),
                                preferred_element_type=jnp.float32)
        acc = p if acc is None else acc + p
    y_ref[...] = acc.reshape(th, n, -1).astype(y_ref.dtype)
    s_ref[0] = jnp.sum(acc, axis=0, keepdims=True)
    ss_ref[0] = jnp.sum(acc * acc, axis=0, keepdims=True)


def _bn_relu_kernel(y_ref, sc_ref, sh_ref, o_ref):
    cout, th2, w, n = o_ref.shape
    for h in range(th2):
        yt = jnp.transpose(y_ref[h], (1, 0)).astype(jnp.float32)
        o = jnp.maximum(yt * sc_ref[...] + sh_ref[...], 0.0)
        o_ref[:, h] = o.reshape(cout, w, n)


def kernel(x_nchw, w_oihw, gamma, beta, eps=1e-5):
    N, Cin, H, W = x_nchw.shape
    Cout, Cin2, KH, KW = w_oihw.shape
    assert Cin2 == Cin and KH == 3 and KW == 3

    K = Cin * W
    Lout = Cout * W
    th = 4 if H % 4 == 0 else 1
    T = H // th

    x_t = jnp.transpose(x_nchw, (1, 2, 3, 0))

    wt = jnp.transpose(w_oihw, (2, 3, 1, 0)).astype(jnp.float32)
    wi = jnp.arange(W)[None, :, None]
    wo = jnp.arange(W)[None, None, :]
    dx = jnp.arange(KW)[:, None, None]
    S = (wi - wo + 1 == dx).astype(jnp.float32)
    B = jnp.einsum('xwv,dxco->dcwov', S, wt).reshape(KH, K, Lout)
    B = B.astype(jnp.bfloat16)

    cparams = pltpu.CompilerParams(
        dimension_semantics=("parallel",),
        vmem_limit_bytes=96 * 1024 * 1024,
    )

    conv_y, psum, psumsq = pl.pallas_call(
        _conv_stats_kernel,
        grid=(T,),
        out_shape=(
            jax.ShapeDtypeStruct((H, N, Lout), jnp.bfloat16),
            jax.ShapeDtypeStruct((T, 1, Lout), jnp.float32),
            jax.ShapeDtypeStruct((T, 1, Lout), jnp.float32),
        ),
        in_specs=[
            pl.BlockSpec((Cin, th, W, N), lambda i: (0, i, 0, 0)),
            pl.BlockSpec((Cin, 1, W, N),
                         lambda i: (0, jnp.maximum(i * th - 1, 0), 0, 0)),
            pl.BlockSpec((Cin, 1, W, N),
                         lambda i: (0, jnp.minimum((i + 1) * th, H - 1), 0, 0)),
            pl.BlockSpec((KH, K, Lout), lambda i: (0, 0, 0)),
        ],
        out_specs=(
            pl.BlockSpec((th, N, Lout), lambda i: (i, 0, 0)),
            pl.BlockSpec((1, 1, Lout), lambda i: (i, 0, 0)),
            pl.BlockSpec((1, 1, Lout), lambda i: (i, 0, 0)),
        ),
        compiler_params=cparams,
        cost_estimate=pl.CostEstimate(
            flops=2 * H * N * KH * K * Lout,
            transcendentals=0,
            bytes_accessed=4 * Cin * H * W * N + 2 * (KH * K * Lout
                           + H * N * Lout) + 8 * T * Lout,
        ),
    )(x_t, x_t, x_t, B)

    count = N * H * W
    lane_sum = jnp.sum(psum, axis=(0, 1))
    lane_sumsq = jnp.sum(psumsq, axis=(0, 1))
    ch_sum = lane_sum.reshape(Cout, W).sum(axis=1)
    ch_sumsq = lane_sumsq.reshape(Cout, W).sum(axis=1)
    mean = ch_sum / count
    var = jnp.maximum(ch_sumsq / count - mean * mean, 0.0)
    inv_std = jax.lax.rsqrt(var + eps)
    scale_c = gamma.astype(jnp.float32) * inv_std
    shift_c = beta.astype(jnp.float32) - mean * scale_c
    scale_r = jnp.repeat(scale_c, W).reshape(Lout, 1)
    shift_r = jnp.repeat(shift_c, W).reshape(Lout, 1)

    th2 = 8 if H % 8 == 0 else 1
    out_t = pl.pallas_call(
        _bn_relu_kernel,
        grid=(H // th2,),
        out_shape=jax.ShapeDtypeStruct((Cout, H, W, N), jnp.float32),
        in_specs=[
            pl.BlockSpec((th2, N, Lout), lambda i: (i, 0, 0)),
            pl.BlockSpec((Lout, 1), lambda i: (0, 0)),
            pl.BlockSpec((Lout, 1), lambda i: (0, 0)),
        ],
        out_specs=pl.BlockSpec((Cout, th2, W, N), lambda i: (0, i, 0, 0)),
        compiler_params=cparams,
        cost_estimate=pl.CostEstimate(
            flops=2 * H * N * Lout,
            transcendentals=0,
            bytes_accessed=6 * H * N * Lout,
        ),
    )(conv_y, scale_r, shift_r)

    return jnp.transpose(out_t, (3, 0, 1, 2))


# speedup vs baseline: 2.8461x; 1.0785x over previous
import jax
import jax.numpy as jnp
from jax.experimental import pallas as pl
from jax.experimental.pallas import tpu as pltpu


def _conv_stats_kernel(xm_ref, xu_ref, xd_ref, b_ref, y_ref, s_ref, ss_ref):
    cin, th, w, n = xm_ref.shape
    k = cin * w
    i = pl.program_id(0)
    t = pl.num_programs(0)
    xm = xm_ref[...].astype(jnp.bfloat16)
    up = (jnp.where(i == 0, 0.0, 1.0) * xu_ref[...]).astype(jnp.bfloat16)
    dn = (jnp.where(i == t - 1, 0.0, 1.0) * xd_ref[...]).astype(jnp.bfloat16)
    cols = [up.reshape(k, n)]
    for h in range(th):
        cols.append(xm[:, h].reshape(k, n))
    cols.append(dn.reshape(k, n))
    a = jnp.concatenate(cols, axis=1)
    acc = None
    for dy in range(3):
        lhs = jax.lax.slice(a, (0, dy * n), (k, (dy + th) * n))
        p = jax.lax.dot_general(lhs, b_ref[dy], (((0,), (0,)), ((), ())),
                                preferred_element_type=jnp.float32)
        acc = p if acc is None else acc + p
    y_ref[...] = acc.reshape(th, n, -1).astype(y_ref.dtype)
    s_ref[0] = jnp.transpose(jnp.sum(acc, axis=0, keepdims=True), (1, 0))
    ss_ref[0] = jnp.transpose(jnp.sum(acc * acc, axis=0, keepdims=True), (1, 0))


def _make_bn_relu_kernel(inv_count, eps):
    def _bn_relu_kernel(y_ref, ps_ref, pss_ref, g_ref, b_ref, o_ref):
        cout, th2, w, n = o_ref.shape
        lane_s = jnp.sum(ps_ref[...], axis=0)
        lane_ss = jnp.sum(pss_ref[...], axis=0)
        ch_s = jnp.sum(lane_s.reshape(cout, w, 1), axis=1)
        ch_ss = jnp.sum(lane_ss.reshape(cout, w, 1), axis=1)
        mean = ch_s * inv_count
        var = jnp.maximum(ch_ss * inv_count - mean * mean, 0.0)
        inv_std = jax.lax.rsqrt(var + eps)
        scale_c = g_ref[...] * inv_std
        shift_c = b_ref[...] - mean * scale_c
        scale_r = jnp.broadcast_to(scale_c[:, None, :],
                                   (cout, w, 1)).reshape(cout * w, 1)
        shift_r = jnp.broadcast_to(shift_c[:, None, :],
                                   (cout, w, 1)).reshape(cout * w, 1)
        for h in range(th2):
            yt = jnp.transpose(y_ref[h], (1, 0)).astype(jnp.float32)
            o = jnp.maximum(yt * scale_r + shift_r, 0.0)
            o_ref[:, h] = o.reshape(cout, w, n)
    return _bn_relu_kernel


def kernel(x_nchw, w_oihw, gamma, beta, eps=1e-5):
    N, Cin, H, W = x_nchw.shape
    Cout, Cin2, KH, KW = w_oihw.shape
    assert Cin2 == Cin and KH == 3 and KW == 3

    K = Cin * W
    Lout = Cout * W
    th = 4 if H % 4 == 0 else 1
    T = H // th

    x_t = jnp.transpose(x_nchw, (1, 2, 3, 0))

    wt = jnp.transpose(w_oihw, (2, 3, 1, 0)).astype(jnp.float32)
    wi = jnp.arange(W)[None, :, None]
    wo = jnp.arange(W)[None, None, :]
    dx = jnp.arange(KW)[:, None, None]
    S = (wi - wo + 1 == dx).astype(jnp.float32)
    B = jnp.sum(S[None, :, None, :, None, :] * wt[:, :, :, None, :, None],
                axis=1).reshape(KH, K, Lout)
    B = B.astype(jnp.bfloat16)

    cparams = pltpu.CompilerParams(
        dimension_semantics=("parallel",),
        vmem_limit_bytes=96 * 1024 * 1024,
    )

    conv_y, psum, psumsq = pl.pallas_call(
        _conv_stats_kernel,
        grid=(T,),
        out_shape=(
            jax.ShapeDtypeStruct((H, N, Lout), jnp.bfloat16),
            jax.ShapeDtypeStruct((T, Lout, 1), jnp.float32),
            jax.ShapeDtypeStruct((T, Lout, 1), jnp.float32),
        ),
        in_specs=[
            pl.BlockSpec((Cin, th, W, N), lambda i: (0, i, 0, 0)),
            pl.BlockSpec((Cin, 1, W, N),
                         lambda i: (0, jnp.maximum(i * th - 1, 0), 0, 0)),
            pl.BlockSpec((Cin, 1, W, N),
                         lambda i: (0, jnp.minimum((i + 1) * th, H - 1), 0, 0)),
            pl.BlockSpec((KH, K, Lout), lambda i: (0, 0, 0)),
        ],
        out_specs=(
            pl.BlockSpec((th, N, Lout), lambda i: (i, 0, 0)),
            pl.BlockSpec((1, Lout, 1), lambda i: (i, 0, 0)),
            pl.BlockSpec((1, Lout, 1), lambda i: (i, 0, 0)),
        ),
        compiler_params=cparams,
        cost_estimate=pl.CostEstimate(
            flops=2 * H * N * KH * K * Lout,
            transcendentals=0,
            bytes_accessed=4 * Cin * H * W * N + 2 * (KH * K * Lout
                           + H * N * Lout) + 8 * T * Lout,
        ),
    )(x_t, x_t, x_t, B)

    gamma_c = gamma.astype(jnp.float32).reshape(Cout, 1)
    beta_c = beta.astype(jnp.float32).reshape(Cout, 1)
    th2 = 8 if H % 8 == 0 else 1
    out_t = pl.pallas_call(
        _make_bn_relu_kernel(1.0 / (N * H * W), eps),
        grid=(H // th2,),
        out_shape=jax.ShapeDtypeStruct((Cout, H, W, N), jnp.float32),
        in_specs=[
            pl.BlockSpec((th2, N, Lout), lambda i: (i, 0, 0)),
            pl.BlockSpec((T, Lout, 1), lambda i: (0, 0, 0)),
            pl.BlockSpec((T, Lout, 1), lambda i: (0, 0, 0)),
            pl.BlockSpec((Cout, 1), lambda i: (0, 0)),
            pl.BlockSpec((Cout, 1), lambda i: (0, 0)),
        ],
        out_specs=pl.BlockSpec((Cout, th2, W, N), lambda i: (0, i, 0, 0)),
        compiler_params=cparams,
        cost_estimate=pl.CostEstimate(
            flops=2 * H * N * Lout,
            transcendentals=0,
            bytes_accessed=6 * H * N * Lout,
        ),
    )(conv_y, psum, psumsq, gamma_c, beta_c)

    return jnp.transpose(out_t, (3, 0, 1, 2))


# speedup vs baseline: 2.8716x; 1.0090x over previous
import jax
import jax.numpy as jnp
import numpy as np
from jax.experimental import pallas as pl
from jax.experimental.pallas import tpu as pltpu


def _conv_stats_kernel(xm_ref, xu_ref, xd_ref, b_ref, y_ref, s_ref, ss_ref):
    cin, th, w, n = xm_ref.shape
    k = cin * w
    i = pl.program_id(0)
    t = pl.num_programs(0)
    xm = xm_ref[...].astype(jnp.bfloat16)
    up = (jnp.where(i == 0, 0.0, 1.0) * xu_ref[...]).astype(jnp.bfloat16)
    dn = (jnp.where(i == t - 1, 0.0, 1.0) * xd_ref[...]).astype(jnp.bfloat16)
    cols = [up.reshape(k, n)]
    for h in range(th):
        cols.append(xm[:, h].reshape(k, n))
    cols.append(dn.reshape(k, n))
    a = jnp.concatenate(cols, axis=1)
    acc = None
    for dy in range(3):
        lhs = jax.lax.slice(a, (0, dy * n), (k, (dy + th) * n))
        p = jax.lax.dot_general(lhs, b_ref[dy], (((0,), (0,)), ((), ())),
                                preferred_element_type=jnp.float32)
        acc = p if acc is None else acc + p
    y_ref[...] = acc.reshape(th, n, -1).astype(y_ref.dtype)
    s_ref[0] = jnp.transpose(jnp.sum(acc, axis=0, keepdims=True), (1, 0))
    ss_ref[0] = jnp.transpose(jnp.sum(acc * acc, axis=0, keepdims=True), (1, 0))


def _make_bn_relu_kernel(inv_count, eps):
    def _bn_relu_kernel(y_ref, ps_ref, pss_ref, g_ref, b_ref, o_ref):
        cout, th2, w, n = o_ref.shape
        lane_s = jnp.sum(ps_ref[...], axis=0)
        lane_ss = jnp.sum(pss_ref[...], axis=0)
        ch_s = jnp.sum(lane_s.reshape(cout, w, 1), axis=1)
        ch_ss = jnp.sum(lane_ss.reshape(cout, w, 1), axis=1)
        mean = ch_s * inv_count
        var = jnp.maximum(ch_ss * inv_count - mean * mean, 0.0)
        inv_std = jax.lax.rsqrt(var + eps)
        scale_c = jnp.transpose(g_ref[...], (1, 0)) * inv_std
        shift_c = jnp.transpose(b_ref[...], (1, 0)) - mean * scale_c
        scale_r = jnp.broadcast_to(scale_c[:, None, :],
                                   (cout, w, 1)).reshape(cout * w, 1)
        shift_r = jnp.broadcast_to(shift_c[:, None, :],
                                   (cout, w, 1)).reshape(cout * w, 1)
        for h in range(th2):
            yt = jnp.transpose(y_ref[h], (1, 0)).astype(jnp.float32)
            o = jnp.maximum(yt * scale_r + shift_r, 0.0)
            o_ref[:, h] = o.reshape(cout, w, n)
    return _bn_relu_kernel


def kernel(x_nchw, w_oihw, gamma, beta, eps=1e-5):
    N, Cin, H, W = x_nchw.shape
    Cout, Cin2, KH, KW = w_oihw.shape
    assert Cin2 == Cin and KH == 3 and KW == 3

    K = Cin * W
    Lout = Cout * W
    th = 4 if H % 4 == 0 else 1
    T = H // th

    x_t = jnp.transpose(x_nchw, (1, 2, 3, 0))

    wt = jnp.transpose(w_oihw, (2, 3, 1, 0)).astype(jnp.float32)
    S = jnp.asarray((np.arange(W)[None, :, None] - np.arange(W)[None, None, :]
                     + 1 == np.arange(KW)[:, None, None]).astype(np.float32))
    B = jnp.sum(S[None, :, None, :, None, :] * wt[:, :, :, None, :, None],
                axis=1).reshape(KH, K, Lout)
    B = B.astype(jnp.bfloat16)

    cparams = pltpu.CompilerParams(
        dimension_semantics=("parallel",),
        vmem_limit_bytes=96 * 1024 * 1024,
    )

    conv_y, psum, psumsq = pl.pallas_call(
        _conv_stats_kernel,
        grid=(T,),
        out_shape=(
            jax.ShapeDtypeStruct((H, N, Lout), jnp.bfloat16),
            jax.ShapeDtypeStruct((T, Lout, 1), jnp.float32),
            jax.ShapeDtypeStruct((T, Lout, 1), jnp.float32),
        ),
        in_specs=[
            pl.BlockSpec((Cin, th, W, N), lambda i: (0, i, 0, 0)),
            pl.BlockSpec((Cin, 1, W, N),
                         lambda i: (0, jnp.maximum(i * th - 1, 0), 0, 0)),
            pl.BlockSpec((Cin, 1, W, N),
                         lambda i: (0, jnp.minimum((i + 1) * th, H - 1), 0, 0)),
            pl.BlockSpec((KH, K, Lout), lambda i: (0, 0, 0)),
        ],
        out_specs=(
            pl.BlockSpec((th, N, Lout), lambda i: (i, 0, 0)),
            pl.BlockSpec((1, Lout, 1), lambda i: (i, 0, 0)),
            pl.BlockSpec((1, Lout, 1), lambda i: (i, 0, 0)),
        ),
        compiler_params=cparams,
        cost_estimate=pl.CostEstimate(
            flops=2 * H * N * KH * K * Lout,
            transcendentals=0,
            bytes_accessed=4 * Cin * H * W * N + 2 * (KH * K * Lout
                           + H * N * Lout) + 8 * T * Lout,
        ),
    )(x_t, x_t, x_t, B)

    gamma_c = gamma.astype(jnp.float32).reshape(1, Cout)
    beta_c = beta.astype(jnp.float32).reshape(1, Cout)
    th2 = 8 if H % 8 == 0 else 1
    out_t = pl.pallas_call(
        _make_bn_relu_kernel(1.0 / (N * H * W), eps),
        grid=(H // th2,),
        out_shape=jax.ShapeDtypeStruct((Cout, H, W, N), jnp.float32),
        in_specs=[
            pl.BlockSpec((th2, N, Lout), lambda i: (i, 0, 0)),
            pl.BlockSpec((T, Lout, 1), lambda i: (0, 0, 0)),
            pl.BlockSpec((T, Lout, 1), lambda i: (0, 0, 0)),
            pl.BlockSpec((1, Cout), lambda i: (0, 0)),
            pl.BlockSpec((1, Cout), lambda i: (0, 0)),
        ],
        out_specs=pl.BlockSpec((Cout, th2, W, N), lambda i: (0, i, 0, 0)),
        compiler_params=cparams,
        cost_estimate=pl.CostEstimate(
            flops=2 * H * N * Lout,
            transcendentals=0,
            bytes_accessed=6 * H * N * Lout,
        ),
    )(conv_y, psum, psumsq, gamma_c, beta_c)

    return jnp.transpose(out_t, (3, 0, 1, 2))


# speedup vs baseline: 2.8962x; 1.0086x over previous
import jax
import jax.numpy as jnp
import numpy as np
from jax.experimental import pallas as pl
from jax.experimental.pallas import tpu as pltpu


def _conv_stats_kernel(xm_ref, xu_ref, xd_ref, b_ref, y_ref, s_ref, ss_ref):
    cin, th, w, n = xm_ref.shape
    k = cin * w
    i = pl.program_id(0)
    t = pl.num_programs(0)
    xm = xm_ref[...].astype(jnp.bfloat16)
    up = (jnp.where(i == 0, 0.0, 1.0) * xu_ref[...]).astype(jnp.bfloat16)
    dn = (jnp.where(i == t - 1, 0.0, 1.0) * xd_ref[...]).astype(jnp.bfloat16)
    cols = [up.reshape(k, n)]
    for h in range(th):
        cols.append(xm[:, h].reshape(k, n))
    cols.append(dn.reshape(k, n))
    a = jnp.concatenate(cols, axis=1)
    acc = None
    for dy in range(3):
        lhs = jax.lax.slice(a, (0, dy * n), (k, (dy + th) * n))
        p = jax.lax.dot_general(lhs, b_ref[dy], (((0,), (0,)), ((), ())),
                                preferred_element_type=jnp.float32)
        acc = p if acc is None else acc + p
    y_ref[...] = acc.reshape(th, n, -1).astype(y_ref.dtype)
    s_ref[0] = jnp.transpose(jnp.sum(acc, axis=0, keepdims=True), (1, 0))
    ss_ref[0] = jnp.transpose(jnp.sum(acc * acc, axis=0, keepdims=True), (1, 0))


def _make_bn_relu_kernel(inv_count, eps):
    def _bn_relu_kernel(y_ref, ps_ref, pss_ref, g_ref, b_ref, o_ref):
        cout, th2, w, n = o_ref.shape
        lane_s = jnp.sum(ps_ref[...], axis=0)
        lane_ss = jnp.sum(pss_ref[...], axis=0)
        ch_s = jnp.sum(lane_s.reshape(cout, w, 1), axis=1)
        ch_ss = jnp.sum(lane_ss.reshape(cout, w, 1), axis=1)
        mean = ch_s * inv_count
        var = jnp.maximum(ch_ss * inv_count - mean * mean, 0.0)
        inv_std = jax.lax.rsqrt(var + eps)
        scale_c = jnp.transpose(g_ref[...], (1, 0)) * inv_std
        shift_c = jnp.transpose(b_ref[...], (1, 0)) - mean * scale_c
        scale_r = jnp.broadcast_to(scale_c[:, None, :],
                                   (cout, w, 1)).reshape(cout * w, 1)
        shift_r = jnp.broadcast_to(shift_c[:, None, :],
                                   (cout, w, 1)).reshape(cout * w, 1)
        for h in range(th2):
            yt = jnp.transpose(y_ref[h], (1, 0)).astype(jnp.float32)
            o = jnp.maximum(yt * scale_r + shift_r, 0.0)
            o_ref[:, h] = o.reshape(cout, w, n)
    return _bn_relu_kernel


def kernel(x_nchw, w_oihw, gamma, beta, eps=1e-5):
    N, Cin, H, W = x_nchw.shape
    Cout, Cin2, KH, KW = w_oihw.shape
    assert Cin2 == Cin and KH == 3 and KW == 3

    K = Cin * W
    Lout = Cout * W
    th = 8 if H % 8 == 0 else (4 if H % 4 == 0 else 1)
    T = H // th

    x_t = jnp.transpose(x_nchw, (1, 2, 3, 0))

    wt = jnp.transpose(w_oihw, (2, 3, 1, 0)).astype(jnp.float32)
    S = jnp.asarray((np.arange(W)[None, :, None] - np.arange(W)[None, None, :]
                     + 1 == np.arange(KW)[:, None, None]).astype(np.float32))
    B = jnp.sum(S[None, :, None, :, None, :] * wt[:, :, :, None, :, None],
                axis=1).reshape(KH, K, Lout)
    B = B.astype(jnp.bfloat16)

    cparams = pltpu.CompilerParams(
        dimension_semantics=("parallel",),
        vmem_limit_bytes=96 * 1024 * 1024,
    )

    conv_y, psum, psumsq = pl.pallas_call(
        _conv_stats_kernel,
        grid=(T,),
        out_shape=(
            jax.ShapeDtypeStruct((H, N, Lout), jnp.bfloat16),
            jax.ShapeDtypeStruct((T, Lout, 1), jnp.float32),
            jax.ShapeDtypeStruct((T, Lout, 1), jnp.float32),
        ),
        in_specs=[
            pl.BlockSpec((Cin, th, W, N), lambda i: (0, i, 0, 0)),
            pl.BlockSpec((Cin, 1, W, N),
                         lambda i: (0, jnp.maximum(i * th - 1, 0), 0, 0)),
            pl.BlockSpec((Cin, 1, W, N),
                         lambda i: (0, jnp.minimum((i + 1) * th, H - 1), 0, 0)),
            pl.BlockSpec((KH, K, Lout), lambda i: (0, 0, 0)),
        ],
        out_specs=(
            pl.BlockSpec((th, N, Lout), lambda i: (i, 0, 0)),
            pl.BlockSpec((1, Lout, 1), lambda i: (i, 0, 0)),
            pl.BlockSpec((1, Lout, 1), lambda i: (i, 0, 0)),
        ),
        compiler_params=cparams,
        cost_estimate=pl.CostEstimate(
            flops=2 * H * N * KH * K * Lout,
            transcendentals=0,
            bytes_accessed=4 * Cin * H * W * N + 2 * (KH * K * Lout
                           + H * N * Lout) + 8 * T * Lout,
        ),
    )(x_t, x_t, x_t, B)

    gamma_c = gamma.astype(jnp.float32).reshape(1, Cout)
    beta_c = beta.astype(jnp.float32).reshape(1, Cout)
    th2 = 8 if H % 8 == 0 else 1
    out_t = pl.pallas_call(
        _make_bn_relu_kernel(1.0 / (N * H * W), eps),
        grid=(H // th2,),
        out_shape=jax.ShapeDtypeStruct((Cout, H, W, N), jnp.float32),
        in_specs=[
            pl.BlockSpec((th2, N, Lout), lambda i: (i, 0, 0)),
            pl.BlockSpec((T, Lout, 1), lambda i: (0, 0, 0)),
            pl.BlockSpec((T, Lout, 1), lambda i: (0, 0, 0)),
            pl.BlockSpec((1, Cout), lambda i: (0, 0)),
            pl.BlockSpec((1, Cout), lambda i: (0, 0)),
        ],
        out_specs=pl.BlockSpec((Cout, th2, W, N), lambda i: (0, i, 0, 0)),
        compiler_params=cparams,
        cost_estimate=pl.CostEstimate(
            flops=2 * H * N * Lout,
            transcendentals=0,
            bytes_accessed=6 * H * N * Lout,
        ),
    )(conv_y, psum, psumsq, gamma_c, beta_c)

    return jnp.transpose(out_t, (3, 0, 1, 2))
